# baseline (speedup 1.0000x reference)
import functools

import jax
import jax.numpy as jnp
from jax import lax
from jax.experimental import pallas as pl
from jax.experimental.pallas import tpu as pltpu
from jax.experimental.pallas import tpu_sc as plsc

_VOCAB = 1000


@functools.lru_cache(maxsize=None)
def _make_sc_onehot(n_rows: int, vocab: int):
    info = plsc.get_sparse_core_info()
    num_cores, num_subcores, lanes = (
        info.num_cores, info.num_subcores, info.num_lanes)
    n_workers = num_cores * num_subcores
    rows_per_w = n_rows // n_workers
    assert rows_per_w * n_workers == n_rows

    batch_rows = 64
    while rows_per_w % batch_rows:
        batch_rows //= 2
    n_batches = rows_per_w // batch_rows
    buf_words = batch_rows * vocab

    mesh = plsc.VectorSubcoreMesh(core_axis_name="c", subcore_axis_name="s")

    @functools.partial(
        pl.kernel,
        mesh=mesh,
        out_type=jax.ShapeDtypeStruct((n_rows * vocab,), jnp.float32),
        scratch_types=[
            pltpu.VMEM((rows_per_w,), jnp.int32),
            pltpu.VMEM((buf_words,), jnp.float32),
        ],
        compiler_params=pltpu.CompilerParams(needs_layout_passes=False),
    )
    def onehot(idx_hbm, out_hbm, idx_v, buf):
        wid = lax.axis_index("s") * num_cores + lax.axis_index("c")
        row0 = wid * rows_per_w
        pltpu.sync_copy(idx_hbm.at[pl.ds(row0, rows_per_w)], idx_v)

        zeros16 = jnp.zeros((lanes,), jnp.float32)
        ones16 = jnp.ones((lanes,), jnp.float32)
        lane = lax.iota(jnp.int32, lanes)

        def zero_body(i, c):
            buf[pl.ds(i * lanes, lanes)] = zeros16
            return c

        lax.fori_loop(0, buf_words // lanes, zero_body, 0)

        def batch_body(b, c):
            for j in range(batch_rows // lanes):
                rows = idx_v[pl.ds(b * batch_rows + j * lanes, lanes)]
                pos = (j * lanes + lane) * vocab + rows
                plsc.store_scatter(buf, [pos], ones16)
            pltpu.sync_copy(
                buf,
                out_hbm.at[pl.ds((row0 + b * batch_rows) * vocab, buf_words)],
            )
            for j in range(batch_rows // lanes):
                rows = idx_v[pl.ds(b * batch_rows + j * lanes, lanes)]
                pos = (j * lanes + lane) * vocab + rows
                plsc.store_scatter(buf, [pos], zeros16)
            return c

        lax.fori_loop(0, n_batches, batch_body, 0)

    return onehot


def kernel(x):
    n0, n1 = x.shape
    idx = x.reshape(-1).astype(jnp.int32)
    out = _make_sc_onehot(n0 * n1, _VOCAB)(idx)
    return out.reshape(n0, n1, _VOCAB)
